# baseline (speedup 1.0000x reference)
import jax
import jax.numpy as jnp
from jax import lax
from jax.experimental import pallas as pl
from jax.experimental.pallas import tpu as pltpu
from jax.experimental.pallas import tpu_sc as plsc

N_CORES = 2
N_SUBCORES = 16
NW = N_CORES * N_SUBCORES
BATCH = 16384
D_MODEL = 128
BPW = BATCH // NW
CHUNK = 128
NCH = BPW // CHUNK
NTOT = 2 * NCH
NBUF = 6

REL_PAD = 512
REL_BLK = 1024


def _ent_body(qe_hbm, oe_hbm, ent_hbm, out_qe, out_oe,
              idx_v, rows_v, gsem, ssem, isem):
    wid = lax.axis_index("s") * N_CORES + lax.axis_index("c")
    base = wid * BPW
    idx_srcs = (qe_hbm, oe_hbm)
    outs = (out_qe, out_oe)

    icps = [pltpu.async_copy(
        idx_srcs[t // NCH].at[pl.ds(base + (t % NCH) * CHUNK, CHUNK)],
        idx_v.at[t], isem) for t in range(NTOT)]
    idx_ready = [False] * NTOT

    def gather(t, slot):
        if not idx_ready[t]:
            icps[t].wait()
            idx_ready[t] = True
        return pltpu.async_copy(
            ent_hbm.at[idx_v.at[t]],
            rows_v.at[pl.ds(slot * CHUNK, CHUNK)], gsem)

    NU = NTOT // 2

    def ugather(u):
        return (gather(2 * u, (u % 3) * 2), gather(2 * u + 1, (u % 3) * 2 + 1))

    def uscatter(u):
        p, j = divmod(u, NCH // 2)
        return pltpu.async_copy(
            rows_v.at[pl.ds((u % 3) * 2 * CHUNK, 2 * CHUNK)],
            outs[p].at[pl.ds(base + j * 2 * CHUNK, 2 * CHUNK)], ssem)

    gcps = {u: ugather(u) for u in range(2)}
    scps = {}
    drained = set()
    for u in range(NU):
        for cp in gcps[u]:
            cp.wait()
        scps[u] = uscatter(u)
        nxt = u + 2
        if nxt < NU:
            if u >= 1:
                scps[u - 1].wait()
                drained.add(u - 1)
            gcps[nxt] = ugather(nxt)
    for u in range(NU):
        if u not in drained:
            scps[u].wait()


def _rel_body(idx_ref, rel_cat_ref, out_ref):
    idx_col = jnp.transpose(idx_ref[0], (1, 0))
    ks = lax.broadcasted_iota(jnp.int32, (REL_BLK, REL_PAD), 1)
    onehot = (idx_col == ks).astype(jnp.float32)
    cat = jnp.dot(onehot, rel_cat_ref[...],
                  preferred_element_type=jnp.float32)
    out_ref[...] = cat[:, :D_MODEL] + cat[:, D_MODEL:]


@jax.jit
def kernel(query_entities, query_relations, obj_entities, ent_table, rel_table):
    out = jax.ShapeDtypeStruct((BATCH, D_MODEL), jnp.float32)

    mesh = plsc.VectorSubcoreMesh(core_axis_name="c", subcore_axis_name="s")
    ent_call = pl.kernel(
        _ent_body,
        out_type=(out, out),
        mesh=mesh,
        scratch_types=[
            pltpu.VMEM((NTOT, CHUNK), jnp.int32),
            pltpu.VMEM((NBUF * CHUNK, D_MODEL), jnp.float32),
            pltpu.SemaphoreType.DMA,
            pltpu.SemaphoreType.DMA,
            pltpu.SemaphoreType.DMA,
        ],
    )
    out_qe, out_oe = ent_call(query_entities.astype(jnp.int32),
                              obj_entities.astype(jnp.int32),
                              ent_table)

    rel_pad = jnp.pad(rel_table, ((0, REL_PAD - rel_table.shape[0]), (0, 0)))
    rel_bits = lax.bitcast_convert_type(rel_pad, jnp.uint32)
    rel_hi = lax.bitcast_convert_type(
        rel_bits & jnp.uint32(0xFFFF0000), jnp.float32)
    rel_cat = jnp.concatenate([rel_hi, rel_pad - rel_hi], axis=1)
    out_qr = pl.pallas_call(
        _rel_body,
        grid=(BATCH // REL_BLK,),
        in_specs=[
            pl.BlockSpec((1, 1, REL_BLK), lambda i: (i, 0, 0)),
            pl.BlockSpec((REL_PAD, 2 * D_MODEL), lambda i: (0, 0)),
        ],
        out_specs=pl.BlockSpec((REL_BLK, D_MODEL), lambda i: (i, 0)),
        out_shape=out,
    )(query_relations.astype(jnp.int32).reshape(BATCH // REL_BLK, 1, REL_BLK),
      rel_cat)

    return (out_qe, out_qr, out_oe)


# speedup vs baseline: 1.0764x; 1.0764x over previous
import jax
import jax.numpy as jnp
from jax import lax
from jax.experimental import pallas as pl
from jax.experimental.pallas import tpu as pltpu
from jax.experimental.pallas import tpu_sc as plsc

N_CORES = 2
N_SUBCORES = 16
NW = N_CORES * N_SUBCORES
BATCH = 16384
D_MODEL = 128
BPW = BATCH // NW
CHUNK = 128
NCH = BPW // CHUNK
NTOT = 2 * NCH
NBUF = 6

REL_PAD = 512
REL_BLK = 4096


def _ent_body(qe_hbm, oe_hbm, ent_hbm, out_qe, out_oe,
              idx_v, rows_v, gsem, ssem, isem):
    wid = lax.axis_index("s") * N_CORES + lax.axis_index("c")
    base = wid * BPW
    idx_srcs = (qe_hbm, oe_hbm)
    outs = (out_qe, out_oe)

    icps = [pltpu.async_copy(
        idx_srcs[t // NCH].at[pl.ds(base + (t % NCH) * CHUNK, CHUNK)],
        idx_v.at[t], isem) for t in range(NTOT)]
    idx_ready = [False] * NTOT

    def gather(t, slot):
        if not idx_ready[t]:
            icps[t].wait()
            idx_ready[t] = True
        return pltpu.async_copy(
            ent_hbm.at[idx_v.at[t]],
            rows_v.at[pl.ds(slot * CHUNK, CHUNK)], gsem)

    NU = NTOT // 2

    def ugather(u):
        return (gather(2 * u, (u % 3) * 2), gather(2 * u + 1, (u % 3) * 2 + 1))

    def uscatter(u):
        p, j = divmod(u, NCH // 2)
        return pltpu.async_copy(
            rows_v.at[pl.ds((u % 3) * 2 * CHUNK, 2 * CHUNK)],
            outs[p].at[pl.ds(base + j * 2 * CHUNK, 2 * CHUNK)], ssem)

    gcps = {u: ugather(u) for u in range(2)}
    scps = {}
    drained = set()
    for u in range(NU):
        for cp in gcps[u]:
            cp.wait()
        scps[u] = uscatter(u)
        nxt = u + 2
        if nxt < NU:
            if u >= 1:
                scps[u - 1].wait()
                drained.add(u - 1)
            gcps[nxt] = ugather(nxt)
    for u in range(NU):
        if u not in drained:
            scps[u].wait()


def _rel_body(idx_ref, rel_cat_ref, out_ref):
    idx_col = jnp.transpose(idx_ref[0], (1, 0))
    ks = lax.broadcasted_iota(jnp.int32, (REL_BLK, REL_PAD), 1)
    onehot = (idx_col == ks).astype(jnp.float32)
    cat = jnp.dot(onehot, rel_cat_ref[...],
                  preferred_element_type=jnp.float32)
    out_ref[...] = cat[:, :D_MODEL] + cat[:, D_MODEL:]


@jax.jit
def kernel(query_entities, query_relations, obj_entities, ent_table, rel_table):
    out = jax.ShapeDtypeStruct((BATCH, D_MODEL), jnp.float32)

    mesh = plsc.VectorSubcoreMesh(core_axis_name="c", subcore_axis_name="s")
    ent_call = pl.kernel(
        _ent_body,
        out_type=(out, out),
        mesh=mesh,
        scratch_types=[
            pltpu.VMEM((NTOT, CHUNK), jnp.int32),
            pltpu.VMEM((NBUF * CHUNK, D_MODEL), jnp.float32),
            pltpu.SemaphoreType.DMA,
            pltpu.SemaphoreType.DMA,
            pltpu.SemaphoreType.DMA,
        ],
    )
    out_qe, out_oe = ent_call(query_entities.astype(jnp.int32),
                              obj_entities.astype(jnp.int32),
                              ent_table)

    rel_pad = jnp.pad(rel_table, ((0, REL_PAD - rel_table.shape[0]), (0, 0)))
    rel_bits = lax.bitcast_convert_type(rel_pad, jnp.uint32)
    rel_hi = lax.bitcast_convert_type(
        rel_bits & jnp.uint32(0xFFFF0000), jnp.float32)
    rel_cat = jnp.concatenate([rel_hi, rel_pad - rel_hi], axis=1)
    out_qr = pl.pallas_call(
        _rel_body,
        grid=(BATCH // REL_BLK,),
        in_specs=[
            pl.BlockSpec((1, 1, REL_BLK), lambda i: (i, 0, 0)),
            pl.BlockSpec((REL_PAD, 2 * D_MODEL), lambda i: (0, 0)),
        ],
        out_specs=pl.BlockSpec((REL_BLK, D_MODEL), lambda i: (i, 0)),
        out_shape=out,
    )(query_relations.astype(jnp.int32).reshape(BATCH // REL_BLK, 1, REL_BLK),
      rel_cat)

    return (out_qe, out_qr, out_oe)


# speedup vs baseline: 1.0857x; 1.0087x over previous
import jax
import jax.numpy as jnp
from jax import lax
from jax.experimental import pallas as pl
from jax.experimental.pallas import tpu as pltpu
from jax.experimental.pallas import tpu_sc as plsc

N_CORES = 2
N_SUBCORES = 16
NW = N_CORES * N_SUBCORES
BATCH = 16384
D_MODEL = 128
BPW = BATCH // NW
CHUNK = 128
NCH = BPW // CHUNK
NTOT = 2 * NCH
NBUF = 6

REL_PAD = 512
REL_BLK = 2048


def _ent_body(qe_hbm, oe_hbm, ent_hbm, out_qe, out_oe,
              idx_v, rows_v, gsem, ssem, isem):
    wid = lax.axis_index("s") * N_CORES + lax.axis_index("c")
    base = wid * BPW
    idx_srcs = (qe_hbm, oe_hbm)
    outs = (out_qe, out_oe)

    icps = [pltpu.async_copy(
        idx_srcs[t // NCH].at[pl.ds(base + (t % NCH) * CHUNK, CHUNK)],
        idx_v.at[t], isem) for t in range(NTOT)]
    idx_ready = [False] * NTOT

    def gather(t, slot):
        if not idx_ready[t]:
            icps[t].wait()
            idx_ready[t] = True
        return pltpu.async_copy(
            ent_hbm.at[idx_v.at[t]],
            rows_v.at[pl.ds(slot * CHUNK, CHUNK)], gsem)

    NU = NTOT // 2

    def ugather(u):
        return (gather(2 * u, (u % 3) * 2), gather(2 * u + 1, (u % 3) * 2 + 1))

    def uscatter(u):
        p, j = divmod(u, NCH // 2)
        return pltpu.async_copy(
            rows_v.at[pl.ds((u % 3) * 2 * CHUNK, 2 * CHUNK)],
            outs[p].at[pl.ds(base + j * 2 * CHUNK, 2 * CHUNK)], ssem)

    gcps = {u: ugather(u) for u in range(2)}
    scps = {}
    drained = set()
    for u in range(NU):
        for cp in gcps[u]:
            cp.wait()
        scps[u] = uscatter(u)
        nxt = u + 2
        if nxt < NU:
            if u >= 1:
                scps[u - 1].wait()
                drained.add(u - 1)
            gcps[nxt] = ugather(nxt)
    for u in range(NU):
        if u not in drained:
            scps[u].wait()


def _rel_body(idx_ref, rel_cat_ref, out_ref):
    idx_col = jnp.transpose(idx_ref[0], (1, 0))
    ks = lax.broadcasted_iota(jnp.int32, (REL_BLK, REL_PAD), 1)
    onehot = (idx_col == ks).astype(jnp.float32)
    cat = jnp.dot(onehot, rel_cat_ref[...],
                  preferred_element_type=jnp.float32)
    out_ref[...] = cat[:, :D_MODEL] + cat[:, D_MODEL:]


@jax.jit
def kernel(query_entities, query_relations, obj_entities, ent_table, rel_table):
    out = jax.ShapeDtypeStruct((BATCH, D_MODEL), jnp.float32)

    mesh = plsc.VectorSubcoreMesh(core_axis_name="c", subcore_axis_name="s")
    ent_call = pl.kernel(
        _ent_body,
        out_type=(out, out),
        mesh=mesh,
        scratch_types=[
            pltpu.VMEM((NTOT, CHUNK), jnp.int32),
            pltpu.VMEM((NBUF * CHUNK, D_MODEL), jnp.float32),
            pltpu.SemaphoreType.DMA,
            pltpu.SemaphoreType.DMA,
            pltpu.SemaphoreType.DMA,
        ],
    )
    out_qe, out_oe = ent_call(query_entities.astype(jnp.int32),
                              obj_entities.astype(jnp.int32),
                              ent_table)

    rel_pad = jnp.pad(rel_table, ((0, REL_PAD - rel_table.shape[0]), (0, 0)))
    rel_bits = lax.bitcast_convert_type(rel_pad, jnp.uint32)
    rel_hi = lax.bitcast_convert_type(
        rel_bits & jnp.uint32(0xFFFF0000), jnp.float32)
    rel_cat = jnp.concatenate([rel_hi, rel_pad - rel_hi], axis=1)
    out_qr = pl.pallas_call(
        _rel_body,
        grid=(BATCH // REL_BLK,),
        in_specs=[
            pl.BlockSpec((1, 1, REL_BLK), lambda i: (i, 0, 0)),
            pl.BlockSpec((REL_PAD, 2 * D_MODEL), lambda i: (0, 0)),
        ],
        out_specs=pl.BlockSpec((REL_BLK, D_MODEL), lambda i: (i, 0)),
        out_shape=out,
    )(query_relations.astype(jnp.int32).reshape(BATCH // REL_BLK, 1, REL_BLK),
      rel_cat)

    return (out_qe, out_qr, out_oe)
